# baseline (speedup 1.0000x reference)
import jax
import jax.numpy as jnp
from jax.experimental import pallas as pl
from jax.experimental.pallas import tpu as pltpu
from jax.experimental.pallas import tpu_sc as plsc

_BM = 1024
_WIN = 128


def _tc_encode_body(x_ref, wb_ref, bb_ref, wh_ref, bh_ref, st_ref,
                    ind_ref, sn_ref, wb_scr, wh_scr, snt_scr, sn_scr):
    i = pl.program_id(0)
    kk = st_ref.shape[0]
    bf = jnp.bfloat16

    @pl.when(i == 0)
    def _():
        st = st_ref[...]
        n = jnp.sqrt(jnp.sum(st * st, axis=1, keepdims=True))
        sn = st / jnp.maximum(n, 1e-12)
        sn_scr[...] = sn
        sn_ref[...] = sn
        snt_scr[...] = sn.astype(bf).T
        wb_scr[...] = wb_ref[...].astype(bf)
        wh_scr[...] = wh_ref[...].astype(bf)

    h = jnp.dot(x_ref[...].astype(bf), wb_scr[...],
                preferred_element_type=jnp.float32)
    h = jnp.maximum(h + bb_ref[...], 0.0)
    z = jnp.dot(h.astype(bf), wh_scr[...],
                preferred_element_type=jnp.float32)
    z = z + bh_ref[...]
    zn = z / jnp.maximum(jnp.sqrt(jnp.sum(z * z, axis=1, keepdims=True)),
                         1e-12)
    s = jnp.dot(zn.astype(bf), snt_scr[...],
                preferred_element_type=jnp.float32)
    ind = jnp.argmax(s, axis=1)
    ind_ref[0, 0, :] = ind.astype(jnp.int32)


def kernel(x, W_body, b_body, W_head, b_head, abs_states):
    bsz, din = x.shape
    feat = W_body.shape[1]
    d = W_head.shape[1]
    k = abs_states.shape[0]
    bm = _BM
    nb = bsz // bm

    bb2 = b_body.reshape(1, feat)
    bh2 = b_head.reshape(1, d)

    ind3, sn = pl.pallas_call(
        _tc_encode_body,
        grid=(nb,),
        in_specs=[
            pl.BlockSpec((bm, din), lambda i: (i, 0)),
            pl.BlockSpec((din, feat), lambda i: (0, 0)),
            pl.BlockSpec((1, feat), lambda i: (0, 0)),
            pl.BlockSpec((feat, d), lambda i: (0, 0)),
            pl.BlockSpec((1, d), lambda i: (0, 0)),
            pl.BlockSpec((k, d), lambda i: (0, 0)),
        ],
        out_specs=[
            pl.BlockSpec((1, 1, bm), lambda i: (i, 0, 0)),
            pl.BlockSpec((k, d), lambda i: (0, 0)),
        ],
        out_shape=[
            jax.ShapeDtypeStruct((nb, 1, bm), jnp.int32),
            jax.ShapeDtypeStruct((k, d), jnp.float32),
        ],
        scratch_shapes=[
            pltpu.VMEM((din, feat), jnp.bfloat16),
            pltpu.VMEM((feat, d), jnp.bfloat16),
            pltpu.VMEM((d, k), jnp.bfloat16),
            pltpu.VMEM((k, d), jnp.float32),
        ],
    )(x, W_body, bb2, W_head, bh2, abs_states)

    ind = ind3.reshape(1, bsz)

    vector_mesh = plsc.VectorSubcoreMesh(
        core_axis_name="core", subcore_axis_name="subcore")
    n_tiles = vector_mesh.num_cores * vector_mesh.num_subcores
    per = bsz // n_tiles

    @pl.kernel(out_type=jax.ShapeDtypeStruct((bsz, d), jnp.float32),
               mesh=vector_mesh,
               scratch_types=[pltpu.VMEM((per,), jnp.int32),
                              pltpu.VMEM((per, d), jnp.float32),
                              pltpu.SemaphoreType.DMA])
    def _sc_gather(sn_hbm, i_hbm, o_hbm, idx_v, rows_v, sem):
        c = jax.lax.axis_index("core")
        sc = jax.lax.axis_index("subcore")
        base = (c * vector_mesh.num_subcores + sc) * per
        pltpu.sync_copy(i_hbm.at[0, pl.ds(base, per)], idx_v)
        pltpu.async_copy(sn_hbm.at[idx_v], rows_v, sem).wait()
        pltpu.sync_copy(rows_v, o_hbm.at[pl.ds(base, per)])

    return _sc_gather(sn, ind)


# speedup vs baseline: 1.0006x; 1.0006x over previous
import jax
import jax.numpy as jnp
from jax.experimental import pallas as pl
from jax.experimental.pallas import tpu as pltpu
from jax.experimental.pallas import tpu_sc as plsc

_BM = 1024


def _tc_encode_body(x_ref, wb_ref, bb_ref, wh_ref, bh_ref, st_ref,
                    ind_ref, sn_ref, wb_scr, wh_scr, snt_scr, sn_scr):
    i = pl.program_id(0)
    bf = jnp.bfloat16

    @pl.when(i == 0)
    def _():
        st = st_ref[...]
        n = jnp.sqrt(jnp.sum(st * st, axis=1, keepdims=True))
        sn = st / jnp.maximum(n, 1e-12)
        sn_scr[...] = sn
        sn_ref[...] = sn
        snt_scr[...] = sn.astype(bf).T
        wb_scr[...] = wb_ref[...].astype(bf)
        wh_scr[...] = wh_ref[...].astype(bf)

    h = jnp.dot(x_ref[...].astype(bf), wb_scr[...],
                preferred_element_type=jnp.float32)
    h = jnp.maximum(h + bb_ref[...], 0.0)
    z = jnp.dot(h.astype(bf), wh_scr[...],
                preferred_element_type=jnp.float32)
    z = z + bh_ref[...]
    zn = z / jnp.maximum(jnp.sqrt(jnp.sum(z * z, axis=1, keepdims=True)),
                         1e-12)
    s = jnp.dot(zn.astype(bf), snt_scr[...],
                preferred_element_type=jnp.float32)
    ind = jnp.argmax(s, axis=1)
    ind_ref[0, 0, :] = ind.astype(jnp.int32)


def kernel(x, W_body, b_body, W_head, b_head, abs_states):
    bsz, din = x.shape
    feat = W_body.shape[1]
    d = W_head.shape[1]
    k = abs_states.shape[0]
    bm = _BM
    nb = bsz // bm

    bb2 = b_body.reshape(1, feat)
    bh2 = b_head.reshape(1, d)

    ind3, sn = pl.pallas_call(
        _tc_encode_body,
        grid=(nb,),
        in_specs=[
            pl.BlockSpec((bm, din), lambda i: (i, 0)),
            pl.BlockSpec((din, feat), lambda i: (0, 0)),
            pl.BlockSpec((1, feat), lambda i: (0, 0)),
            pl.BlockSpec((feat, d), lambda i: (0, 0)),
            pl.BlockSpec((1, d), lambda i: (0, 0)),
            pl.BlockSpec((k, d), lambda i: (0, 0)),
        ],
        out_specs=[
            pl.BlockSpec((1, 1, bm), lambda i: (i, 0, 0)),
            pl.BlockSpec((k, d), lambda i: (0, 0)),
        ],
        out_shape=[
            jax.ShapeDtypeStruct((nb, 1, bm), jnp.int32),
            jax.ShapeDtypeStruct((k, d), jnp.float32),
        ],
        scratch_shapes=[
            pltpu.VMEM((din, feat), jnp.bfloat16),
            pltpu.VMEM((feat, d), jnp.bfloat16),
            pltpu.VMEM((d, k), jnp.bfloat16),
            pltpu.VMEM((k, d), jnp.float32),
        ],
    )(x, W_body, bb2, W_head, bh2, abs_states)

    ind = ind3.reshape(1, bsz)

    vector_mesh = plsc.VectorSubcoreMesh(
        core_axis_name="core", subcore_axis_name="subcore")
    n_tiles = vector_mesh.num_cores * vector_mesh.num_subcores
    per = bsz // n_tiles

    @pl.kernel(out_type=jax.ShapeDtypeStruct((bsz, d), jnp.float32),
               mesh=vector_mesh,
               scratch_types=[pltpu.VMEM((per,), jnp.int32),
                              pltpu.VMEM((per, d), jnp.float32),
                              pltpu.SemaphoreType.DMA])
    def _sc_gather(sn_hbm, i_hbm, o_hbm, idx_v, rows_v, sem):
        c = jax.lax.axis_index("core")
        sc = jax.lax.axis_index("subcore")
        base = (c * vector_mesh.num_subcores + sc) * per
        pltpu.sync_copy(i_hbm.at[0, pl.ds(base, per)], idx_v)
        pltpu.async_copy(sn_hbm.at[idx_v], rows_v, sem).wait()
        pltpu.sync_copy(rows_v, o_hbm.at[pl.ds(base, per)])

    return _sc_gather(sn, ind)
